# Initial kernel scaffold; baseline (speedup 1.0000x reference)
import jax
import jax.numpy as jnp
from jax.experimental import pallas as pl


def kernel(x, ca1_W1, ca1_b1, ca1_W2, ca1_b2, pa1_W1, pa1_b1, pa1_W2, pa1_b2, fc1_W, ca2_W1, ca2_b1, ca2_W2, ca2_b2, pa2_W1, pa2_b1, pa2_W2, pa2_b2, lin_W, bn_gamma, bn_beta, bn_mean, bn_var):
    raise NotImplementedError("write your pallas kernel here")


# speedup vs baseline: 1.7413x; 1.7413x over previous
import functools

import jax
import jax.numpy as jnp
from jax.experimental import pallas as pl

_M, _S, _C, _F = 50000, 32, 10, 64
_MB = 2048
_GRID = (_M + _MB - 1) // _MB


def _paca(slabs, cW1T, cb1, cW2T, cb2, pW1T, pb1, pW2T, pb2):
    cmax = functools.reduce(jnp.maximum, slabs)
    ymax = jnp.concatenate(
        [jnp.max(sl, axis=0, keepdims=True) for sl in slabs], axis=0)
    cz = jnp.maximum(jnp.dot(cW1T, cmax, preferred_element_type=jnp.float32) + cb1, 0.0)
    cw = jax.nn.sigmoid(jnp.dot(cW2T, cz, preferred_element_type=jnp.float32) + cb2)
    pz = jnp.maximum(jnp.dot(pW1T, ymax, preferred_element_type=jnp.float32) + pb1, 0.0)
    pw = jax.nn.sigmoid(jnp.dot(pW2T, pz, preferred_element_type=jnp.float32) + pb2)
    return cw, pw


def _body(xt_ref, caW1T, cab1, caW2T, cab2, paW1T, pab1, paW2T, pab2, fcWT,
          ca2W1T, ca2b1, ca2W2T, ca2b2, pa2W1T, pa2b1, pa2W2T, pa2b2,
          linWT, shift, out_ref):
    xs = [xt_ref[s * _C:(s + 1) * _C, :] for s in range(_S)]
    cw1, pw1 = _paca(xs, caW1T[...], cab1[...], caW2T[...], cab2[...],
                     paW1T[...], pab1[...], paW2T[...], pab2[...])
    fcT = fcWT[...]
    out1 = []
    for s in range(_S):
        o = xs[s] * cw1 * pw1[s:s + 1, :]
        cat = jnp.concatenate([xs[s], o], axis=0)
        out1.append(jnp.dot(fcT, cat, preferred_element_type=jnp.float32))
    cw2, pw2 = _paca(out1, ca2W1T[...], ca2b1[...], ca2W2T[...], ca2b2[...],
                     pa2W1T[...], pa2b1[...], pa2W2T[...], pa2b2[...])
    lT = linWT[...]
    sh = shift[...]
    hmax = None
    for s in range(_S):
        o2 = out1[s] * cw2 * pw2[s:s + 1, :]
        h = jnp.maximum(jnp.dot(lT, o2, preferred_element_type=jnp.float32) + sh, 0.0)
        hmax = h if hmax is None else jnp.maximum(hmax, h)
    out_ref[:, :] = hmax


def kernel(x, ca1_W1, ca1_b1, ca1_W2, ca1_b2, pa1_W1, pa1_b1, pa1_W2, pa1_b2,
           fc1_W, ca2_W1, ca2_b1, ca2_W2, ca2_b2, pa2_W1, pa2_b1, pa2_W2,
           pa2_b2, lin_W, bn_gamma, bn_beta, bn_mean, bn_var):
    xt = x.reshape(_M, _S * _C).T

    scale = bn_gamma * jax.lax.rsqrt(bn_var + 1e-3)
    shift = (bn_beta - bn_mean * scale).reshape(_F, 1)
    linWT = (lin_W * scale[None, :]).T

    small = (
        ca1_W1.T, ca1_b1.reshape(_C, 1), ca1_W2.T, ca1_b2.reshape(_C, 1),
        pa1_W1.T, pa1_b1.reshape(_S, 1), pa1_W2.T, pa1_b2.reshape(_S, 1),
        fc1_W.T,
        ca2_W1.T, ca2_b1.reshape(_C, 1), ca2_W2.T, ca2_b2.reshape(_C, 1),
        pa2_W1.T, pa2_b1.reshape(_S, 1), pa2_W2.T, pa2_b2.reshape(_S, 1),
        linWT, shift,
    )

    res = pl.pallas_call(
        _body,
        grid=(_GRID,),
        in_specs=[pl.BlockSpec((_S * _C, _MB), lambda i: (0, i))] + [
            pl.BlockSpec(a.shape, lambda i: (0, 0)) for a in small],
        out_specs=pl.BlockSpec((_F, _MB), lambda i: (0, i)),
        out_shape=jax.ShapeDtypeStruct((_F, _M), jnp.float32),
    )(xt, *small)

    return res.T.reshape(_M, 1, _F)
